# Initial kernel scaffold; baseline (speedup 1.0000x reference)
import jax
import jax.numpy as jnp
from jax.experimental import pallas as pl


def kernel(w1, b1, g1, be1, w2, b2, g2, be2, wfc, bfc, wh, bh, x):
    raise NotImplementedError("write your pallas kernel here")


# speedup vs baseline: 6.3956x; 6.3956x over previous
import functools

import jax
import jax.numpy as jnp
from jax.experimental import pallas as pl
from jax.experimental.pallas import tpu as pltpu

_EPS = 1e-5
_VMEM_LIMIT = 64 * 1024 * 1024
_F32 = jnp.float32


def _iota_eq(rows, cols, mod):
    r = jax.lax.broadcasted_iota(jnp.int32, (rows, cols), 0)
    c = jax.lax.broadcasted_iota(jnp.int32, (rows, cols), 1)
    return (r % mod == c % mod).astype(_F32) if mod != cols else \
        (r % mod == c).astype(_F32)


def _stage1(x_ref, w_ref, b_ref, out_ref, st_ref, a0, a1, a2, a3):
    accs = (a0, a1, a2, a3)
    for ck in range(8):
        r0 = 128 * ck
        nr = 128 if ck < 7 else 62
        acc = None
        for t in range(9):
            i, j = divmod(t, 3)
            off = i * 32 + j
            m = jnp.dot(x_ref[r0 + off:r0 + off + nr, :],
                        w_ref[t * 128:(t + 1) * 128, :],
                        preferred_element_type=_F32)
            acc = m if acc is None else acc + m
        for k in range(4):
            accs[k][r0:r0 + nr, :] = acc[:, 128 * k:128 * (k + 1)]
    for k in range(4):
        accs[k][958:1024, :] = jnp.zeros((66, 128), _F32)

    for hc in range(2):
        for k in range(4):
            e = jnp.maximum(accs[k][pl.ds(512 * hc, 256, 2), :],
                            accs[k][pl.ds(512 * hc + 1, 256, 2), :])
            w4 = e.reshape(8, 2, 16, 128)
            hm = jnp.maximum(w4[:, 0], w4[:, 1]).reshape(128, 128)
            hm = hm + b_ref[:, 128 * k:128 * (k + 1)]
            out_ref[128 * hc:128 * hc + 128, 128 * k:128 * (k + 1)] = hm

    pv = out_ref[0:240, :]
    ri = jax.lax.broadcasted_iota(jnp.int32, (240, 1), 0)
    pm = pv * (ri % 16 < 15).astype(_F32)
    fold = _iota_eq(512, 16, 16)
    st_ref[0, 0:1, :] = jnp.dot(jnp.sum(pm, axis=0, keepdims=True), fold,
                                preferred_element_type=_F32)
    st_ref[0, 1:2, :] = jnp.dot(jnp.sum(pm * pm, axis=0, keepdims=True), fold,
                                preferred_element_type=_F32)


def _stage2(x_ref, sin_ref, g_ref, be_ref, w_ref, b_ref, out_ref, st_ref,
            a0, a1, *, groups, inv_m):
    s = jnp.sum(sin_ref[...], axis=0)
    mean = s[0:1, :] * inv_m
    var = jnp.maximum(s[1:2, :] * inv_m - mean * mean, 0.0)
    scale_c = jax.lax.rsqrt(var + _EPS) * g_ref[...]
    shift_c = be_ref[...] - mean * scale_c
    expand = _iota_eq(16, 128, 16)
    scale = jnp.dot(scale_c, expand, preferred_element_type=_F32)
    shift = jnp.dot(shift_c, expand, preferred_element_type=_F32)

    ri = jax.lax.broadcasted_iota(jnp.int32, (64, 1), 0)
    mask = ((ri % 8 < 6) & (ri // 8 < 6)).astype(_F32)
    fold = _iota_eq(128, 32, 32)
    accs = (a0, a1)
    st0 = None
    st1 = None
    for q in range(groups):
        a = jnp.maximum(x_ref[:, q * 128:(q + 1) * 128] * scale + shift, 0.0)
        acc = None
        for t in range(9):
            i, j = divmod(t, 3)
            off = i * 16 + j
            m = jnp.dot(a[off:off + 222, :], w_ref[t * 128:(t + 1) * 128, :],
                        preferred_element_type=_F32)
            acc = m if acc is None else acc + m
        for k in range(2):
            accs[k][0:222, :] = acc[:, 128 * k:128 * (k + 1)]
            accs[k][222:256, :] = jnp.zeros((34, 128), _F32)
            e = jnp.maximum(accs[k][pl.ds(0, 128, 2), :],
                            accs[k][pl.ds(1, 128, 2), :])
            w4 = e.reshape(8, 2, 8, 128)
            hm = jnp.maximum(w4[:, 0], w4[:, 1]).reshape(64, 128)
            hm = hm + b_ref[:, 128 * k:128 * (k + 1)]
            out_ref[:, q * 256 + 128 * k:q * 256 + 128 * (k + 1)] = hm
            pm = hm * mask
            s0 = jnp.dot(jnp.sum(pm, axis=0, keepdims=True), fold,
                         preferred_element_type=_F32)
            s1 = jnp.dot(jnp.sum(pm * pm, axis=0, keepdims=True), fold,
                         preferred_element_type=_F32)
            st0 = s0 if st0 is None else st0 + s0
            st1 = s1 if st1 is None else st1 + s1
    st_ref[0, 0:1, :] = st0
    st_ref[0, 1:2, :] = st1


def _stage3(x_ref, sin_ref, g_ref, be_ref, wfc_ref, bfc_ref, wh_ref, bh_ref,
            out_ref, *, inv_m, segments):
    s = jnp.sum(sin_ref[...], axis=0)
    mean = s[0:1, :] * inv_m
    var = jnp.maximum(s[1:2, :] * inv_m - mean * mean, 0.0)
    scale_c = jax.lax.rsqrt(var + _EPS) * g_ref[...]
    shift_c = be_ref[...] - mean * scale_c
    expand = _iota_eq(32, 1152, 32)
    scale = jnp.dot(scale_c, expand, preferred_element_type=_F32)
    shift = jnp.dot(shift_c, expand, preferred_element_type=_F32)
    act = jnp.maximum(x_ref[...] * scale + shift, 0.0)
    sh = jnp.dot(act, wfc_ref[...], preferred_element_type=_F32)
    sh = jnp.maximum(sh + bfc_ref[...], 0.0)
    z = jnp.dot(sh, wh_ref[...], preferred_element_type=_F32) + bh_ref[...]
    for s0, c in segments:
        zt = z[:, s0:s0 + c]
        m = jnp.max(zt, axis=1, keepdims=True)
        lse = jnp.log(jnp.sum(jnp.exp(zt - m), axis=1, keepdims=True))
        out_ref[:, s0:s0 + c] = zt - m - lse


_SEGMENTS = ((0, 10), (10, 5))
_CTOT = 15


@jax.jit
def _forward(w1, b1, g1, be1, w2, b2, g2, be2, wfc, bfc, wh, bh, x):
    n = x.shape[0]
    g2s, be2s = g2[:, :32], be2[:, :32]

    xp = jnp.concatenate([x, jnp.zeros((n, 1, 32, 32), _F32)], axis=1)
    x_l = jnp.transpose(xp, (2, 3, 0, 1)).reshape(1024, n * 4)
    eye32, eye8 = jnp.eye(32, dtype=_F32), jnp.eye(8, dtype=_F32)
    w1bd = jnp.concatenate([
        jnp.kron(eye32, jnp.concatenate(
            [w1[3 * t:3 * t + 3, :], jnp.zeros((1, 16), _F32)], axis=0))
        for t in range(9)], axis=0)
    w2bd = jnp.concatenate([
        jnp.kron(eye8, w2[16 * t:16 * t + 16, :]) for t in range(9)],
        axis=0)
    b1l = jnp.tile(b1, (1, 32))
    b2l = jnp.tile(b2, (1, 8))

    nb1 = n // 32
    pooled1, st1 = pl.pallas_call(
        _stage1,
        grid=(nb1,),
        in_specs=[
            pl.BlockSpec((1024, 128), lambda i: (0, i)),
            pl.BlockSpec((1152, 512), lambda i: (0, 0)),
            pl.BlockSpec((1, 512), lambda i: (0, 0)),
        ],
        out_specs=[
            pl.BlockSpec((256, 512), lambda i: (0, i)),
            pl.BlockSpec((1, 2, 16), lambda i: (i, 0, 0)),
        ],
        out_shape=[
            jax.ShapeDtypeStruct((256, n * 16), _F32),
            jax.ShapeDtypeStruct((nb1, 2, 16), _F32),
        ],
        scratch_shapes=[pltpu.VMEM((1024, 128), _F32)] * 4,
        compiler_params=pltpu.CompilerParams(
            dimension_semantics=("parallel",),
            vmem_limit_bytes=_VMEM_LIMIT),
    )(x_l, w1bd, b1l)

    gr2 = 8
    nb2 = n // (8 * gr2)
    pooled2, st2 = pl.pallas_call(
        functools.partial(_stage2, groups=gr2, inv_m=1.0 / float(n * 225)),
        grid=(nb2,),
        in_specs=[
            pl.BlockSpec((256, 128 * gr2), lambda i: (0, i)),
            pl.BlockSpec((nb1, 2, 16), lambda i: (0, 0, 0)),
            pl.BlockSpec((1, 16), lambda i: (0, 0)),
            pl.BlockSpec((1, 16), lambda i: (0, 0)),
            pl.BlockSpec((1152, 256), lambda i: (0, 0)),
            pl.BlockSpec((1, 256), lambda i: (0, 0)),
        ],
        out_specs=[
            pl.BlockSpec((64, 256 * gr2), lambda i: (0, i)),
            pl.BlockSpec((1, 2, 32), lambda i: (i, 0, 0)),
        ],
        out_shape=[
            jax.ShapeDtypeStruct((64, n * 32), _F32),
            jax.ShapeDtypeStruct((nb2, 2, 32), _F32),
        ],
        scratch_shapes=[pltpu.VMEM((256, 128), _F32)] * 2,
        compiler_params=pltpu.CompilerParams(
            dimension_semantics=("parallel",),
            vmem_limit_bytes=_VMEM_LIMIT),
    )(pooled1, st1, g1, be1, w2bd, b2l)

    p4 = pooled2.reshape(8, 8, n, 32)[:6, :6]
    x3 = jnp.transpose(p4, (2, 0, 1, 3)).reshape(n, 1152)

    tb3 = min(256, n)
    logp = pl.pallas_call(
        functools.partial(_stage3, inv_m=1.0 / float(n * 36),
                          segments=_SEGMENTS),
        grid=(n // tb3,),
        in_specs=[
            pl.BlockSpec((tb3, 1152), lambda i: (i, 0)),
            pl.BlockSpec((nb2, 2, 32), lambda i: (0, 0, 0)),
            pl.BlockSpec((1, 32), lambda i: (0, 0)),
            pl.BlockSpec((1, 32), lambda i: (0, 0)),
            pl.BlockSpec((1152, 32), lambda i: (0, 0)),
            pl.BlockSpec((1, 32), lambda i: (0, 0)),
            pl.BlockSpec((32, _CTOT), lambda i: (0, 0)),
            pl.BlockSpec((1, _CTOT), lambda i: (0, 0)),
        ],
        out_specs=pl.BlockSpec((tb3, _CTOT), lambda i: (i, 0)),
        out_shape=jax.ShapeDtypeStruct((n, _CTOT), _F32),
        compiler_params=pltpu.CompilerParams(
            dimension_semantics=("parallel",),
            vmem_limit_bytes=_VMEM_LIMIT),
    )(x3, st2, g2s, be2s, wfc, bfc, wh, bh)

    return [logp[:, s:s + c] for s, c in _SEGMENTS]


def kernel(w1, b1, g1, be1, w2, b2, g2, be2, wfc, bfc, wh, bh, x):
    return _forward(w1, b1, g1, be1, w2, b2, g2, be2, wfc, bfc, wh, bh, x)


# speedup vs baseline: 6.5548x; 1.0249x over previous
import functools

import jax
import jax.numpy as jnp
from jax.experimental import pallas as pl
from jax.experimental.pallas import tpu as pltpu

_EPS = 1e-5
_VMEM_LIMIT = 64 * 1024 * 1024
_F32 = jnp.float32


def _iota_eq(rows, cols, mod):
    r = jax.lax.broadcasted_iota(jnp.int32, (rows, cols), 0)
    c = jax.lax.broadcasted_iota(jnp.int32, (rows, cols), 1)
    return (r % mod == c % mod).astype(_F32) if mod != cols else \
        (r % mod == c).astype(_F32)


def _stage1(x_ref, w_ref, b_ref, out_ref, st_ref, a0, a1, a2, a3, xb):
    xb[...] = x_ref[...].astype(jnp.bfloat16)
    accs = (a0, a1, a2, a3)
    for ck in range(8):
        r0 = 128 * ck
        nr = 128 if ck < 7 else 62
        acc = None
        for t in range(9):
            i, j = divmod(t, 3)
            off = i * 32 + j
            m = jnp.dot(xb[r0 + off:r0 + off + nr, :],
                        w_ref[t * 128:(t + 1) * 128, :],
                        preferred_element_type=_F32)
            acc = m if acc is None else acc + m
        for k in range(4):
            accs[k][r0:r0 + nr, :] = acc[:, 128 * k:128 * (k + 1)]
    for k in range(4):
        accs[k][958:1024, :] = jnp.zeros((66, 128), _F32)

    for hc in range(2):
        for k in range(4):
            e = jnp.maximum(accs[k][pl.ds(512 * hc, 256, 2), :],
                            accs[k][pl.ds(512 * hc + 1, 256, 2), :])
            w4 = e.reshape(8, 2, 16, 128)
            hm = jnp.maximum(w4[:, 0], w4[:, 1]).reshape(128, 128)
            hm = hm + b_ref[:, 128 * k:128 * (k + 1)]
            out_ref[128 * hc:128 * hc + 128, 128 * k:128 * (k + 1)] = hm

    pv = out_ref[0:240, :]
    ri = jax.lax.broadcasted_iota(jnp.int32, (240, 1), 0)
    pm = pv * (ri % 16 < 15).astype(_F32)
    fold = _iota_eq(512, 16, 16)
    st_ref[0, 0:1, :] = jnp.dot(jnp.sum(pm, axis=0, keepdims=True), fold,
                                preferred_element_type=_F32)
    st_ref[0, 1:2, :] = jnp.dot(jnp.sum(pm * pm, axis=0, keepdims=True), fold,
                                preferred_element_type=_F32)


def _stage2(x_ref, sin_ref, g_ref, be_ref, w_ref, b_ref, out_ref, st_ref,
            a0, a1, *, groups, inv_m):
    s = jnp.sum(sin_ref[...], axis=0)
    mean = s[0:1, :] * inv_m
    var = jnp.maximum(s[1:2, :] * inv_m - mean * mean, 0.0)
    scale_c = jax.lax.rsqrt(var + _EPS) * g_ref[...]
    shift_c = be_ref[...] - mean * scale_c
    expand = _iota_eq(16, 128, 16)
    scale = jnp.dot(scale_c, expand, preferred_element_type=_F32)
    shift = jnp.dot(shift_c, expand, preferred_element_type=_F32)

    ri = jax.lax.broadcasted_iota(jnp.int32, (64, 1), 0)
    mask = ((ri % 8 < 6) & (ri // 8 < 6)).astype(_F32)
    fold = _iota_eq(128, 32, 32)
    accs = (a0, a1)
    st0 = None
    st1 = None
    for q in range(groups):
        a = jnp.maximum(x_ref[:, q * 128:(q + 1) * 128] * scale + shift, 0.0)
        ab = a.astype(jnp.bfloat16)
        acc = None
        for t in range(9):
            i, j = divmod(t, 3)
            off = i * 16 + j
            m = jnp.dot(ab[off:off + 222, :], w_ref[t * 128:(t + 1) * 128, :],
                        preferred_element_type=_F32)
            acc = m if acc is None else acc + m
        for k in range(2):
            accs[k][0:222, :] = acc[:, 128 * k:128 * (k + 1)]
            accs[k][222:256, :] = jnp.zeros((34, 128), _F32)
            e = jnp.maximum(accs[k][pl.ds(0, 128, 2), :],
                            accs[k][pl.ds(1, 128, 2), :])
            w4 = e.reshape(8, 2, 8, 128)
            hm = jnp.maximum(w4[:, 0], w4[:, 1]).reshape(64, 128)
            hm = hm + b_ref[:, 128 * k:128 * (k + 1)]
            out_ref[:, q * 256 + 128 * k:q * 256 + 128 * (k + 1)] = hm
            pm = hm * mask
            s0 = jnp.dot(jnp.sum(pm, axis=0, keepdims=True), fold,
                         preferred_element_type=_F32)
            s1 = jnp.dot(jnp.sum(pm * pm, axis=0, keepdims=True), fold,
                         preferred_element_type=_F32)
            st0 = s0 if st0 is None else st0 + s0
            st1 = s1 if st1 is None else st1 + s1
    st_ref[0, 0:1, :] = st0
    st_ref[0, 1:2, :] = st1


def _stage3(x_ref, sin_ref, g_ref, be_ref, wfc_ref, bfc_ref, wh_ref, bh_ref,
            out_ref, *, inv_m, segments):
    s = jnp.sum(sin_ref[...], axis=0)
    mean = s[0:1, :] * inv_m
    var = jnp.maximum(s[1:2, :] * inv_m - mean * mean, 0.0)
    scale_c = jax.lax.rsqrt(var + _EPS) * g_ref[...]
    shift_c = be_ref[...] - mean * scale_c
    expand = _iota_eq(32, 1152, 32)
    scale = jnp.dot(scale_c, expand, preferred_element_type=_F32)
    shift = jnp.dot(shift_c, expand, preferred_element_type=_F32)
    act = jnp.maximum(x_ref[...] * scale + shift, 0.0)
    sh = jnp.dot(act, wfc_ref[...], preferred_element_type=_F32)
    sh = jnp.maximum(sh + bfc_ref[...], 0.0)
    z = jnp.dot(sh, wh_ref[...], preferred_element_type=_F32) + bh_ref[...]
    for s0, c in segments:
        zt = z[:, s0:s0 + c]
        m = jnp.max(zt, axis=1, keepdims=True)
        lse = jnp.log(jnp.sum(jnp.exp(zt - m), axis=1, keepdims=True))
        out_ref[:, s0:s0 + c] = zt - m - lse


_SEGMENTS = ((0, 10), (10, 5))
_CTOT = 15


@jax.jit
def _forward(w1, b1, g1, be1, w2, b2, g2, be2, wfc, bfc, wh, bh, x):
    n = x.shape[0]
    g2s, be2s = g2[:, :32], be2[:, :32]

    xp = jnp.concatenate([x, jnp.zeros((n, 1, 32, 32), _F32)], axis=1)
    x_l = jnp.transpose(xp, (2, 3, 0, 1)).reshape(1024, n * 4)
    eye32, eye8 = jnp.eye(32, dtype=_F32), jnp.eye(8, dtype=_F32)
    w1bd = jnp.concatenate([
        jnp.kron(eye32, jnp.concatenate(
            [w1[3 * t:3 * t + 3, :], jnp.zeros((1, 16), _F32)], axis=0))
        for t in range(9)], axis=0).astype(jnp.bfloat16)
    w2bd = jnp.concatenate([
        jnp.kron(eye8, w2[16 * t:16 * t + 16, :]) for t in range(9)],
        axis=0).astype(jnp.bfloat16)
    b1l = jnp.tile(b1, (1, 32))
    b2l = jnp.tile(b2, (1, 8))

    nb1 = n // 32
    pooled1, st1 = pl.pallas_call(
        _stage1,
        grid=(nb1,),
        in_specs=[
            pl.BlockSpec((1024, 128), lambda i: (0, i)),
            pl.BlockSpec((1152, 512), lambda i: (0, 0)),
            pl.BlockSpec((1, 512), lambda i: (0, 0)),
        ],
        out_specs=[
            pl.BlockSpec((256, 512), lambda i: (0, i)),
            pl.BlockSpec((1, 2, 16), lambda i: (i, 0, 0)),
        ],
        out_shape=[
            jax.ShapeDtypeStruct((256, n * 16), _F32),
            jax.ShapeDtypeStruct((nb1, 2, 16), _F32),
        ],
        scratch_shapes=[pltpu.VMEM((1024, 128), _F32)] * 4 + [
            pltpu.VMEM((1024, 128), jnp.bfloat16)],
        compiler_params=pltpu.CompilerParams(
            dimension_semantics=("parallel",),
            vmem_limit_bytes=_VMEM_LIMIT),
    )(x_l, w1bd, b1l)

    gr2 = 8
    nb2 = n // (8 * gr2)
    pooled2, st2 = pl.pallas_call(
        functools.partial(_stage2, groups=gr2, inv_m=1.0 / float(n * 225)),
        grid=(nb2,),
        in_specs=[
            pl.BlockSpec((256, 128 * gr2), lambda i: (0, i)),
            pl.BlockSpec((nb1, 2, 16), lambda i: (0, 0, 0)),
            pl.BlockSpec((1, 16), lambda i: (0, 0)),
            pl.BlockSpec((1, 16), lambda i: (0, 0)),
            pl.BlockSpec((1152, 256), lambda i: (0, 0)),
            pl.BlockSpec((1, 256), lambda i: (0, 0)),
        ],
        out_specs=[
            pl.BlockSpec((64, 256 * gr2), lambda i: (0, i)),
            pl.BlockSpec((1, 2, 32), lambda i: (i, 0, 0)),
        ],
        out_shape=[
            jax.ShapeDtypeStruct((64, n * 32), _F32),
            jax.ShapeDtypeStruct((nb2, 2, 32), _F32),
        ],
        scratch_shapes=[pltpu.VMEM((256, 128), _F32)] * 2,
        compiler_params=pltpu.CompilerParams(
            dimension_semantics=("parallel",),
            vmem_limit_bytes=_VMEM_LIMIT),
    )(pooled1, st1, g1, be1, w2bd, b2l)

    p4 = pooled2.reshape(8, 8, n, 32)[:6, :6]
    x3 = jnp.transpose(p4, (2, 0, 1, 3)).reshape(n, 1152)

    tb3 = min(256, n)
    logp = pl.pallas_call(
        functools.partial(_stage3, inv_m=1.0 / float(n * 36),
                          segments=_SEGMENTS),
        grid=(n // tb3,),
        in_specs=[
            pl.BlockSpec((tb3, 1152), lambda i: (i, 0)),
            pl.BlockSpec((nb2, 2, 32), lambda i: (0, 0, 0)),
            pl.BlockSpec((1, 32), lambda i: (0, 0)),
            pl.BlockSpec((1, 32), lambda i: (0, 0)),
            pl.BlockSpec((1152, 32), lambda i: (0, 0)),
            pl.BlockSpec((1, 32), lambda i: (0, 0)),
            pl.BlockSpec((32, _CTOT), lambda i: (0, 0)),
            pl.BlockSpec((1, _CTOT), lambda i: (0, 0)),
        ],
        out_specs=pl.BlockSpec((tb3, _CTOT), lambda i: (i, 0)),
        out_shape=jax.ShapeDtypeStruct((n, _CTOT), _F32),
        compiler_params=pltpu.CompilerParams(
            dimension_semantics=("parallel",),
            vmem_limit_bytes=_VMEM_LIMIT),
    )(x3, st2, g2s, be2s, wfc, bfc, wh, bh)

    return [logp[:, s:s + c] for s, c in _SEGMENTS]


def kernel(w1, b1, g1, be1, w2, b2, g2, be2, wfc, bfc, wh, bh, x):
    return _forward(w1, b1, g1, be1, w2, b2, g2, be2, wfc, bfc, wh, bh, x)


# speedup vs baseline: 7.0082x; 1.0692x over previous
import functools

import jax
import jax.numpy as jnp
from jax.experimental import pallas as pl
from jax.experimental.pallas import tpu as pltpu

_EPS = 1e-5
_VMEM_LIMIT = 64 * 1024 * 1024
_F32 = jnp.float32


def _iota_eq(rows, cols, mod):
    r = jax.lax.broadcasted_iota(jnp.int32, (rows, cols), 0)
    c = jax.lax.broadcasted_iota(jnp.int32, (rows, cols), 1)
    return (r % mod == c % mod).astype(_F32) if mod != cols else \
        (r % mod == c).astype(_F32)


def _stage1(x_ref, w_ref, b_ref, out_ref, st_ref, a0, a1, a2, a3, xb, *,
            groups):
    accs = (a0, a1, a2, a3)
    ri = jax.lax.broadcasted_iota(jnp.int32, (240, 1), 0)
    vmask = (ri % 16 < 15).astype(_F32)
    fold = _iota_eq(512, 16, 16)
    st0 = None
    st1 = None
    for g in range(groups):
        xb[...] = x_ref[:, 128 * g:128 * (g + 1)].astype(jnp.bfloat16)
        for ck in range(8):
            r0 = 128 * ck
            nr = 128 if ck < 7 else 62
            acc = None
            for t in range(9):
                i, j = divmod(t, 3)
                off = i * 32 + j
                m = jnp.dot(xb[r0 + off:r0 + off + nr, :],
                            w_ref[t * 128:(t + 1) * 128, :],
                            preferred_element_type=_F32)
                acc = m if acc is None else acc + m
            for k in range(4):
                accs[k][r0:r0 + nr, :] = acc[:, 128 * k:128 * (k + 1)]
        for k in range(4):
            accs[k][958:1024, :] = jnp.zeros((66, 128), _F32)

        for hc in range(2):
            for k in range(4):
                e = jnp.maximum(accs[k][pl.ds(512 * hc, 256, 2), :],
                                accs[k][pl.ds(512 * hc + 1, 256, 2), :])
                w4 = e.reshape(8, 2, 16, 128)
                hm = jnp.maximum(w4[:, 0], w4[:, 1]).reshape(128, 128)
                hm = hm + b_ref[:, 128 * k:128 * (k + 1)]
                out_ref[128 * hc:128 * hc + 128,
                        512 * g + 128 * k:512 * g + 128 * (k + 1)] = hm

        pv = out_ref[0:240, 512 * g:512 * (g + 1)]
        pm = pv * vmask
        s0 = jnp.dot(jnp.sum(pm, axis=0, keepdims=True), fold,
                     preferred_element_type=_F32)
        s1 = jnp.dot(jnp.sum(pm * pm, axis=0, keepdims=True), fold,
                     preferred_element_type=_F32)
        st0 = s0 if st0 is None else st0 + s0
        st1 = s1 if st1 is None else st1 + s1
    st_ref[0, 0:1, :] = st0
    st_ref[0, 1:2, :] = st1


def _stage2(x_ref, sin_ref, g_ref, be_ref, w_ref, b_ref, out_ref, st_ref,
            a0, a1, *, groups, inv_m):
    s = jnp.sum(sin_ref[...], axis=0)
    mean = s[0:1, :] * inv_m
    var = jnp.maximum(s[1:2, :] * inv_m - mean * mean, 0.0)
    scale_c = jax.lax.rsqrt(var + _EPS) * g_ref[...]
    shift_c = be_ref[...] - mean * scale_c
    expand = _iota_eq(16, 128, 16)
    scale = jnp.dot(scale_c, expand, preferred_element_type=_F32)
    shift = jnp.dot(shift_c, expand, preferred_element_type=_F32)

    ri = jax.lax.broadcasted_iota(jnp.int32, (64, 1), 0)
    mask = ((ri % 8 < 6) & (ri // 8 < 6)).astype(_F32)
    fold = _iota_eq(128, 32, 32)
    accs = (a0, a1)
    st0 = None
    st1 = None
    for q in range(groups):
        a = jnp.maximum(x_ref[:, q * 128:(q + 1) * 128] * scale + shift, 0.0)
        ab = a.astype(jnp.bfloat16)
        acc = None
        for t in range(9):
            i, j = divmod(t, 3)
            off = i * 16 + j
            m = jnp.dot(ab[off:off + 222, :], w_ref[t * 128:(t + 1) * 128, :],
                        preferred_element_type=_F32)
            acc = m if acc is None else acc + m
        for k in range(2):
            accs[k][0:222, :] = acc[:, 128 * k:128 * (k + 1)]
            accs[k][222:256, :] = jnp.zeros((34, 128), _F32)
            e = jnp.maximum(accs[k][pl.ds(0, 128, 2), :],
                            accs[k][pl.ds(1, 128, 2), :])
            w4 = e.reshape(8, 2, 8, 128)
            hm = jnp.maximum(w4[:, 0], w4[:, 1]).reshape(64, 128)
            hm = hm + b_ref[:, 128 * k:128 * (k + 1)]
            out_ref[:, q * 256 + 128 * k:q * 256 + 128 * (k + 1)] = hm
            pm = hm * mask
            s0 = jnp.dot(jnp.sum(pm, axis=0, keepdims=True), fold,
                         preferred_element_type=_F32)
            s1 = jnp.dot(jnp.sum(pm * pm, axis=0, keepdims=True), fold,
                         preferred_element_type=_F32)
            st0 = s0 if st0 is None else st0 + s0
            st1 = s1 if st1 is None else st1 + s1
    st_ref[0, 0:1, :] = st0
    st_ref[0, 1:2, :] = st1


def _stage3(x_ref, sin_ref, g_ref, be_ref, wfc_ref, bfc_ref, wh_ref, bh_ref,
            out_ref, *, inv_m, segments):
    s = jnp.sum(sin_ref[...], axis=0)
    mean = s[0:1, :] * inv_m
    var = jnp.maximum(s[1:2, :] * inv_m - mean * mean, 0.0)
    scale_c = jax.lax.rsqrt(var + _EPS) * g_ref[...]
    shift_c = be_ref[...] - mean * scale_c
    expand = _iota_eq(32, 1152, 32)
    scale = jnp.dot(scale_c, expand, preferred_element_type=_F32)
    shift = jnp.dot(shift_c, expand, preferred_element_type=_F32)
    act = jnp.maximum(x_ref[...] * scale + shift, 0.0)
    sh = jnp.dot(act, wfc_ref[...], preferred_element_type=_F32)
    sh = jnp.maximum(sh + bfc_ref[...], 0.0)
    z = jnp.dot(sh, wh_ref[...], preferred_element_type=_F32) + bh_ref[...]
    for s0, c in segments:
        zt = z[:, s0:s0 + c]
        m = jnp.max(zt, axis=1, keepdims=True)
        lse = jnp.log(jnp.sum(jnp.exp(zt - m), axis=1, keepdims=True))
        out_ref[:, s0:s0 + c] = zt - m - lse


_SEGMENTS = ((0, 10), (10, 5))
_CTOT = 15


@jax.jit
def _forward(w1, b1, g1, be1, w2, b2, g2, be2, wfc, bfc, wh, bh, x):
    n = x.shape[0]
    g2s, be2s = g2[:, :32], be2[:, :32]

    xp = jnp.concatenate([x, jnp.zeros((n, 1, 32, 32), _F32)], axis=1)
    x_l = jnp.transpose(xp, (2, 3, 0, 1)).reshape(1024, n * 4)
    eye32, eye8 = jnp.eye(32, dtype=_F32), jnp.eye(8, dtype=_F32)
    w1bd = jnp.concatenate([
        jnp.kron(eye32, jnp.concatenate(
            [w1[3 * t:3 * t + 3, :], jnp.zeros((1, 16), _F32)], axis=0))
        for t in range(9)], axis=0).astype(jnp.bfloat16)
    w2bd = jnp.concatenate([
        jnp.kron(eye8, w2[16 * t:16 * t + 16, :]) for t in range(9)],
        axis=0).astype(jnp.bfloat16)
    b1l = jnp.tile(b1, (1, 32))
    b2l = jnp.tile(b2, (1, 8))

    gr1 = 4
    nb1 = n // (32 * gr1)
    pooled1, st1 = pl.pallas_call(
        functools.partial(_stage1, groups=gr1),
        grid=(nb1,),
        in_specs=[
            pl.BlockSpec((1024, 128 * gr1), lambda i: (0, i)),
            pl.BlockSpec((1152, 512), lambda i: (0, 0)),
            pl.BlockSpec((1, 512), lambda i: (0, 0)),
        ],
        out_specs=[
            pl.BlockSpec((256, 512 * gr1), lambda i: (0, i)),
            pl.BlockSpec((1, 2, 16), lambda i: (i, 0, 0)),
        ],
        out_shape=[
            jax.ShapeDtypeStruct((256, n * 16), _F32),
            jax.ShapeDtypeStruct((nb1, 2, 16), _F32),
        ],
        scratch_shapes=[pltpu.VMEM((1024, 128), _F32)] * 4 + [
            pltpu.VMEM((1024, 128), jnp.bfloat16)],
        compiler_params=pltpu.CompilerParams(
            dimension_semantics=("parallel",),
            vmem_limit_bytes=_VMEM_LIMIT),
    )(x_l, w1bd, b1l)

    gr2 = 16
    nb2 = n // (8 * gr2)
    pooled2, st2 = pl.pallas_call(
        functools.partial(_stage2, groups=gr2, inv_m=1.0 / float(n * 225)),
        grid=(nb2,),
        in_specs=[
            pl.BlockSpec((256, 128 * gr2), lambda i: (0, i)),
            pl.BlockSpec((nb1, 2, 16), lambda i: (0, 0, 0)),
            pl.BlockSpec((1, 16), lambda i: (0, 0)),
            pl.BlockSpec((1, 16), lambda i: (0, 0)),
            pl.BlockSpec((1152, 256), lambda i: (0, 0)),
            pl.BlockSpec((1, 256), lambda i: (0, 0)),
        ],
        out_specs=[
            pl.BlockSpec((64, 256 * gr2), lambda i: (0, i)),
            pl.BlockSpec((1, 2, 32), lambda i: (i, 0, 0)),
        ],
        out_shape=[
            jax.ShapeDtypeStruct((64, n * 32), _F32),
            jax.ShapeDtypeStruct((nb2, 2, 32), _F32),
        ],
        scratch_shapes=[pltpu.VMEM((256, 128), _F32)] * 2,
        compiler_params=pltpu.CompilerParams(
            dimension_semantics=("parallel",),
            vmem_limit_bytes=_VMEM_LIMIT),
    )(pooled1, st1, g1, be1, w2bd, b2l)

    p4 = pooled2.reshape(8, 8, n, 32)[:6, :6]
    x3 = jnp.transpose(p4, (2, 0, 1, 3)).reshape(n, 1152)

    tb3 = min(512, n)
    logp = pl.pallas_call(
        functools.partial(_stage3, inv_m=1.0 / float(n * 36),
                          segments=_SEGMENTS),
        grid=(n // tb3,),
        in_specs=[
            pl.BlockSpec((tb3, 1152), lambda i: (i, 0)),
            pl.BlockSpec((nb2, 2, 32), lambda i: (0, 0, 0)),
            pl.BlockSpec((1, 32), lambda i: (0, 0)),
            pl.BlockSpec((1, 32), lambda i: (0, 0)),
            pl.BlockSpec((1152, 32), lambda i: (0, 0)),
            pl.BlockSpec((1, 32), lambda i: (0, 0)),
            pl.BlockSpec((32, _CTOT), lambda i: (0, 0)),
            pl.BlockSpec((1, _CTOT), lambda i: (0, 0)),
        ],
        out_specs=pl.BlockSpec((tb3, _CTOT), lambda i: (i, 0)),
        out_shape=jax.ShapeDtypeStruct((n, _CTOT), _F32),
        compiler_params=pltpu.CompilerParams(
            dimension_semantics=("parallel",),
            vmem_limit_bytes=_VMEM_LIMIT),
    )(x3, st2, g2s, be2s, wfc, bfc, wh, bh)

    return [logp[:, s:s + c] for s, c in _SEGMENTS]


def kernel(w1, b1, g1, be1, w2, b2, g2, be2, wfc, bfc, wh, bh, x):
    return _forward(w1, b1, g1, be1, w2, b2, g2, be2, wfc, bfc, wh, bh, x)


# speedup vs baseline: 7.7705x; 1.1088x over previous
import functools

import jax
import jax.numpy as jnp
from jax.experimental import pallas as pl
from jax.experimental.pallas import tpu as pltpu

_EPS = 1e-5
_VMEM_LIMIT = 64 * 1024 * 1024
_F32 = jnp.float32


def _iota_eq(rows, cols, mod):
    r = jax.lax.broadcasted_iota(jnp.int32, (rows, cols), 0)
    c = jax.lax.broadcasted_iota(jnp.int32, (rows, cols), 1)
    return (r % mod == c % mod).astype(_F32) if mod != cols else \
        (r % mod == c).astype(_F32)


def _transpose_in(x_ref, out_ref, *, groups):
    for g in range(groups):
        t = x_ref[96 * g:96 * (g + 1), :].astype(jnp.bfloat16).T
        out_ref[:, 128 * g:128 * g + 96] = t
        out_ref[:, 128 * g + 96:128 * (g + 1)] = jnp.zeros((1024, 32),
                                                           jnp.bfloat16)


def _stage1(x_ref, w_ref, b_ref, out_ref, st_ref, a0, a1, a2, a3, *,
            groups):
    accs = (a0, a1, a2, a3)
    ri = jax.lax.broadcasted_iota(jnp.int32, (240, 1), 0)
    vmask = (ri % 16 < 15).astype(_F32)
    fold = _iota_eq(512, 16, 16)
    st0 = None
    st1 = None
    for g in range(groups):
        for ck in range(8):
            r0 = 128 * ck
            nr = 128 if ck < 7 else 62
            acc = None
            for t in range(9):
                i, j = divmod(t, 3)
                off = i * 32 + j
                m = jnp.dot(x_ref[r0 + off:r0 + off + nr,
                                  128 * g:128 * (g + 1)],
                            w_ref[t * 128:(t + 1) * 128, :],
                            preferred_element_type=_F32)
                acc = m if acc is None else acc + m
            for k in range(4):
                accs[k][r0:r0 + nr, :] = acc[:, 128 * k:128 * (k + 1)]
        for k in range(4):
            accs[k][958:1024, :] = jnp.zeros((66, 128), _F32)

        for hc in range(2):
            for k in range(4):
                e = jnp.maximum(accs[k][pl.ds(512 * hc, 256, 2), :],
                                accs[k][pl.ds(512 * hc + 1, 256, 2), :])
                w4 = e.reshape(8, 2, 16, 128)
                hm = jnp.maximum(w4[:, 0], w4[:, 1]).reshape(128, 128)
                hm = hm + b_ref[:, 128 * k:128 * (k + 1)]
                out_ref[128 * hc:128 * hc + 128,
                        512 * g + 128 * k:512 * g + 128 * (k + 1)] = hm

        pv = out_ref[0:240, 512 * g:512 * (g + 1)]
        pm = pv * vmask
        s0 = jnp.dot(jnp.sum(pm, axis=0, keepdims=True), fold,
                     preferred_element_type=_F32)
        s1 = jnp.dot(jnp.sum(pm * pm, axis=0, keepdims=True), fold,
                     preferred_element_type=_F32)
        st0 = s0 if st0 is None else st0 + s0
        st1 = s1 if st1 is None else st1 + s1
    st_ref[0, 0:1, :] = st0
    st_ref[0, 1:2, :] = st1


def _stage2(x_ref, sin_ref, g_ref, be_ref, w_ref, b_ref, out_ref, st_ref,
            a0, a1, *, groups, inv_m):
    s = jnp.sum(sin_ref[...], axis=0)
    mean = s[0:1, :] * inv_m
    var = jnp.maximum(s[1:2, :] * inv_m - mean * mean, 0.0)
    scale_c = jax.lax.rsqrt(var + _EPS) * g_ref[...]
    shift_c = be_ref[...] - mean * scale_c
    expand = _iota_eq(16, 128, 16)
    scale = jnp.dot(scale_c, expand, preferred_element_type=_F32)
    shift = jnp.dot(shift_c, expand, preferred_element_type=_F32)

    ri = jax.lax.broadcasted_iota(jnp.int32, (64, 1), 0)
    mask = ((ri % 8 < 6) & (ri // 8 < 6)).astype(_F32)
    fold = _iota_eq(128, 32, 32)
    accs = (a0, a1)
    st0 = None
    st1 = None
    for q in range(groups):
        a = jnp.maximum(x_ref[:, q * 128:(q + 1) * 128] * scale + shift, 0.0)
        ab = a.astype(jnp.bfloat16)
        acc = None
        for t in range(9):
            i, j = divmod(t, 3)
            off = i * 16 + j
            m = jnp.dot(ab[off:off + 222, :], w_ref[t * 128:(t + 1) * 128, :],
                        preferred_element_type=_F32)
            acc = m if acc is None else acc + m
        for k in range(2):
            accs[k][0:222, :] = acc[:, 128 * k:128 * (k + 1)]
            accs[k][222:256, :] = jnp.zeros((34, 128), _F32)
            e = jnp.maximum(accs[k][pl.ds(0, 128, 2), :],
                            accs[k][pl.ds(1, 128, 2), :])
            w4 = e.reshape(8, 2, 8, 128)
            hm = jnp.maximum(w4[:, 0], w4[:, 1]).reshape(64, 128)
            hm = hm + b_ref[:, 128 * k:128 * (k + 1)]
            out_ref[:, q * 256 + 128 * k:q * 256 + 128 * (k + 1)] = hm
            pm = hm * mask
            s0 = jnp.dot(jnp.sum(pm, axis=0, keepdims=True), fold,
                         preferred_element_type=_F32)
            s1 = jnp.dot(jnp.sum(pm * pm, axis=0, keepdims=True), fold,
                         preferred_element_type=_F32)
            st0 = s0 if st0 is None else st0 + s0
            st1 = s1 if st1 is None else st1 + s1
    st_ref[0, 0:1, :] = st0
    st_ref[0, 1:2, :] = st1


def _stage3(x_ref, sin_ref, g_ref, be_ref, wfc_ref, bfc_ref, wh_ref, bh_ref,
            out_ref, *, inv_m, segments):
    s = jnp.sum(sin_ref[...], axis=0)
    mean = s[0:1, :] * inv_m
    var = jnp.maximum(s[1:2, :] * inv_m - mean * mean, 0.0)
    scale_c = jax.lax.rsqrt(var + _EPS) * g_ref[...]
    shift_c = be_ref[...] - mean * scale_c
    expand = _iota_eq(32, 1152, 32)
    scale = jnp.dot(scale_c, expand, preferred_element_type=_F32)
    shift = jnp.dot(shift_c, expand, preferred_element_type=_F32)
    act = jnp.maximum(x_ref[...] * scale + shift, 0.0)
    sh = jnp.dot(act, wfc_ref[...], preferred_element_type=_F32)
    sh = jnp.maximum(sh + bfc_ref[...], 0.0)
    z = jnp.dot(sh, wh_ref[...], preferred_element_type=_F32) + bh_ref[...]
    for s0, c in segments:
        zt = z[:, s0:s0 + c]
        m = jnp.max(zt, axis=1, keepdims=True)
        lse = jnp.log(jnp.sum(jnp.exp(zt - m), axis=1, keepdims=True))
        out_ref[:, s0:s0 + c] = zt - m - lse


_SEGMENTS = ((0, 10), (10, 5))
_CTOT = 15


@jax.jit
def _forward(w1, b1, g1, be1, w2, b2, g2, be2, wfc, bfc, wh, bh, x):
    n = x.shape[0]
    g2s, be2s = g2[:, :32], be2[:, :32]

    eye32, eye8 = jnp.eye(32, dtype=_F32), jnp.eye(8, dtype=_F32)
    w1bd = jnp.concatenate([
        jnp.concatenate([jnp.kron(eye32, w1[3 * t:3 * t + 3, :]),
                         jnp.zeros((32, 512), _F32)], axis=0)
        for t in range(9)], axis=0).astype(jnp.bfloat16)
    w2bd = jnp.concatenate([
        jnp.kron(eye8, w2[16 * t:16 * t + 16, :]) for t in range(9)],
        axis=0).astype(jnp.bfloat16)
    b1l = jnp.tile(b1, (1, 32))
    b2l = jnp.tile(b2, (1, 8))

    gr0 = 4
    x_l = pl.pallas_call(
        functools.partial(_transpose_in, groups=gr0),
        grid=(n // (32 * gr0),),
        in_specs=[pl.BlockSpec((96 * gr0, 1024), lambda i: (i, 0))],
        out_specs=pl.BlockSpec((1024, 128 * gr0), lambda i: (0, i)),
        out_shape=jax.ShapeDtypeStruct((1024, n * 4), jnp.bfloat16),
        compiler_params=pltpu.CompilerParams(
            dimension_semantics=("parallel",),
            vmem_limit_bytes=_VMEM_LIMIT),
    )(x.reshape(n * 3, 1024))

    gr1 = 4
    nb1 = n // (32 * gr1)
    pooled1, st1 = pl.pallas_call(
        functools.partial(_stage1, groups=gr1),
        grid=(nb1,),
        in_specs=[
            pl.BlockSpec((1024, 128 * gr1), lambda i: (0, i)),
            pl.BlockSpec((1152, 512), lambda i: (0, 0)),
            pl.BlockSpec((1, 512), lambda i: (0, 0)),
        ],
        out_specs=[
            pl.BlockSpec((256, 512 * gr1), lambda i: (0, i)),
            pl.BlockSpec((1, 2, 16), lambda i: (i, 0, 0)),
        ],
        out_shape=[
            jax.ShapeDtypeStruct((256, n * 16), _F32),
            jax.ShapeDtypeStruct((nb1, 2, 16), _F32),
        ],
        scratch_shapes=[pltpu.VMEM((1024, 128), _F32)] * 4,
        compiler_params=pltpu.CompilerParams(
            dimension_semantics=("parallel",),
            vmem_limit_bytes=_VMEM_LIMIT),
    )(x_l, w1bd, b1l)

    gr2 = 16
    nb2 = n // (8 * gr2)
    pooled2, st2 = pl.pallas_call(
        functools.partial(_stage2, groups=gr2, inv_m=1.0 / float(n * 225)),
        grid=(nb2,),
        in_specs=[
            pl.BlockSpec((256, 128 * gr2), lambda i: (0, i)),
            pl.BlockSpec((nb1, 2, 16), lambda i: (0, 0, 0)),
            pl.BlockSpec((1, 16), lambda i: (0, 0)),
            pl.BlockSpec((1, 16), lambda i: (0, 0)),
            pl.BlockSpec((1152, 256), lambda i: (0, 0)),
            pl.BlockSpec((1, 256), lambda i: (0, 0)),
        ],
        out_specs=[
            pl.BlockSpec((64, 256 * gr2), lambda i: (0, i)),
            pl.BlockSpec((1, 2, 32), lambda i: (i, 0, 0)),
        ],
        out_shape=[
            jax.ShapeDtypeStruct((64, n * 32), _F32),
            jax.ShapeDtypeStruct((nb2, 2, 32), _F32),
        ],
        scratch_shapes=[pltpu.VMEM((256, 128), _F32)] * 2,
        compiler_params=pltpu.CompilerParams(
            dimension_semantics=("parallel",),
            vmem_limit_bytes=_VMEM_LIMIT),
    )(pooled1, st1, g1, be1, w2bd, b2l)

    p4 = pooled2.reshape(8, 8, n, 32)[:6, :6]
    x3 = jnp.transpose(p4, (2, 0, 1, 3)).reshape(n, 1152)

    tb3 = min(512, n)
    logp = pl.pallas_call(
        functools.partial(_stage3, inv_m=1.0 / float(n * 36),
                          segments=_SEGMENTS),
        grid=(n // tb3,),
        in_specs=[
            pl.BlockSpec((tb3, 1152), lambda i: (i, 0)),
            pl.BlockSpec((nb2, 2, 32), lambda i: (0, 0, 0)),
            pl.BlockSpec((1, 32), lambda i: (0, 0)),
            pl.BlockSpec((1, 32), lambda i: (0, 0)),
            pl.BlockSpec((1152, 32), lambda i: (0, 0)),
            pl.BlockSpec((1, 32), lambda i: (0, 0)),
            pl.BlockSpec((32, _CTOT), lambda i: (0, 0)),
            pl.BlockSpec((1, _CTOT), lambda i: (0, 0)),
        ],
        out_specs=pl.BlockSpec((tb3, _CTOT), lambda i: (i, 0)),
        out_shape=jax.ShapeDtypeStruct((n, _CTOT), _F32),
        compiler_params=pltpu.CompilerParams(
            dimension_semantics=("parallel",),
            vmem_limit_bytes=_VMEM_LIMIT),
    )(x3, st2, g2s, be2s, wfc, bfc, wh, bh)

    return [logp[:, s:s + c] for s, c in _SEGMENTS]


def kernel(w1, b1, g1, be1, w2, b2, g2, be2, wfc, bfc, wh, bh, x):
    return _forward(w1, b1, g1, be1, w2, b2, g2, be2, wfc, bfc, wh, bh, x)


# speedup vs baseline: 8.0695x; 1.0385x over previous
import functools

import jax
import jax.numpy as jnp
from jax.experimental import pallas as pl
from jax.experimental.pallas import tpu as pltpu

_EPS = 1e-5
_VMEM_LIMIT = 64 * 1024 * 1024
_F32 = jnp.float32


def _iota_eq(rows, cols, mod):
    r = jax.lax.broadcasted_iota(jnp.int32, (rows, cols), 0)
    c = jax.lax.broadcasted_iota(jnp.int32, (rows, cols), 1)
    return (r % mod == c % mod).astype(_F32) if mod != cols else \
        (r % mod == c).astype(_F32)


def _transpose_in(x_ref, out_ref, *, groups):
    for g in range(groups):
        t = x_ref[96 * g:96 * (g + 1), :].astype(jnp.bfloat16).T
        out_ref[:, 128 * g:128 * g + 96] = t
        out_ref[:, 128 * g + 96:128 * (g + 1)] = jnp.zeros((1024, 32),
                                                           jnp.bfloat16)


def _stage1(x_ref, w_ref, b_ref, out_ref, st_ref, a0, a1, a2, a3, *,
            groups):
    accs = (a0, a1, a2, a3)
    ri = jax.lax.broadcasted_iota(jnp.int32, (240, 1), 0)
    vmask = (ri % 16 < 15).astype(_F32)
    fold = _iota_eq(512, 16, 16)
    st0 = None
    st1 = None
    for g in range(groups):
        for ck in range(8):
            r0 = 128 * ck
            nr = 128 if ck < 7 else 62
            acc = None
            for t in range(9):
                i, j = divmod(t, 3)
                off = i * 32 + j
                m = jnp.dot(x_ref[r0 + off:r0 + off + nr,
                                  128 * g:128 * (g + 1)],
                            w_ref[t * 128:(t + 1) * 128, :],
                            preferred_element_type=_F32)
                acc = m if acc is None else acc + m
            for k in range(4):
                accs[k][r0:r0 + nr, :] = acc[:, 128 * k:128 * (k + 1)]
        for k in range(4):
            accs[k][958:1024, :] = jnp.zeros((66, 128), _F32)

        for hc in range(2):
            for k in range(4):
                e = jnp.maximum(accs[k][pl.ds(512 * hc, 256, 2), :],
                                accs[k][pl.ds(512 * hc + 1, 256, 2), :])
                w4 = e.reshape(8, 2, 16, 128)
                hm = jnp.maximum(w4[:, 0], w4[:, 1]).reshape(128, 128)
                hm = hm + b_ref[:, 128 * k:128 * (k + 1)]
                out_ref[128 * hc:128 * hc + 128,
                        512 * g + 128 * k:512 * g + 128 * (k + 1)] = (
                    hm.astype(jnp.bfloat16))

        pv = out_ref[0:240, 512 * g:512 * (g + 1)].astype(_F32)
        pm = pv * vmask
        s0 = jnp.dot(jnp.sum(pm, axis=0, keepdims=True), fold,
                     preferred_element_type=_F32)
        s1 = jnp.dot(jnp.sum(pm * pm, axis=0, keepdims=True), fold,
                     preferred_element_type=_F32)
        st0 = s0 if st0 is None else st0 + s0
        st1 = s1 if st1 is None else st1 + s1
    st_ref[0, 0:1, :] = st0
    st_ref[0, 1:2, :] = st1


def _stage2(x_ref, sin_ref, g_ref, be_ref, w_ref, b_ref, out_ref, st_ref,
            a0, a1, *, groups, inv_m):
    s = jnp.sum(sin_ref[...], axis=0)
    mean = s[0:1, :] * inv_m
    var = jnp.maximum(s[1:2, :] * inv_m - mean * mean, 0.0)
    scale_c = jax.lax.rsqrt(var + _EPS) * g_ref[...]
    shift_c = be_ref[...] - mean * scale_c
    expand = _iota_eq(16, 128, 16)
    scale = jnp.dot(scale_c, expand, preferred_element_type=_F32)
    shift = jnp.dot(shift_c, expand, preferred_element_type=_F32)

    ri = jax.lax.broadcasted_iota(jnp.int32, (64, 1), 0)
    mask = ((ri % 8 < 6) & (ri // 8 < 6)).astype(_F32)
    fold = _iota_eq(128, 32, 32)
    accs = (a0, a1)
    st0 = None
    st1 = None
    for q in range(groups):
        a = jnp.maximum(x_ref[:, q * 128:(q + 1) * 128] * scale + shift, 0.0)
        ab = a.astype(jnp.bfloat16)
        acc = None
        for t in range(9):
            i, j = divmod(t, 3)
            off = i * 16 + j
            m = jnp.dot(ab[off:off + 222, :], w_ref[t * 128:(t + 1) * 128, :],
                        preferred_element_type=_F32)
            acc = m if acc is None else acc + m
        for k in range(2):
            accs[k][0:222, :] = acc[:, 128 * k:128 * (k + 1)]
            accs[k][222:256, :] = jnp.zeros((34, 128), _F32)
            e = jnp.maximum(accs[k][pl.ds(0, 128, 2), :],
                            accs[k][pl.ds(1, 128, 2), :])
            w4 = e.reshape(8, 2, 8, 128)
            hm = jnp.maximum(w4[:, 0], w4[:, 1]).reshape(64, 128)
            hm = hm + b_ref[:, 128 * k:128 * (k + 1)]
            hm = hm.astype(jnp.bfloat16)
            out_ref[:, q * 256 + 128 * k:q * 256 + 128 * (k + 1)] = hm
            pm = hm.astype(_F32) * mask
            s0 = jnp.dot(jnp.sum(pm, axis=0, keepdims=True), fold,
                         preferred_element_type=_F32)
            s1 = jnp.dot(jnp.sum(pm * pm, axis=0, keepdims=True), fold,
                         preferred_element_type=_F32)
            st0 = s0 if st0 is None else st0 + s0
            st1 = s1 if st1 is None else st1 + s1
    st_ref[0, 0:1, :] = st0
    st_ref[0, 1:2, :] = st1


def _stage3(x_ref, sin_ref, g_ref, be_ref, wfc_ref, bfc_ref, wh_ref, bh_ref,
            out_ref, *, inv_m, segments):
    s = jnp.sum(sin_ref[...], axis=0)
    mean = s[0:1, :] * inv_m
    var = jnp.maximum(s[1:2, :] * inv_m - mean * mean, 0.0)
    scale_c = jax.lax.rsqrt(var + _EPS) * g_ref[...]
    shift_c = be_ref[...] - mean * scale_c
    expand = _iota_eq(32, 1152, 32)
    scale = jnp.dot(scale_c, expand, preferred_element_type=_F32)
    shift = jnp.dot(shift_c, expand, preferred_element_type=_F32)
    act = jnp.maximum(x_ref[...] * scale + shift, 0.0)
    sh = jnp.dot(act, wfc_ref[...], preferred_element_type=_F32)
    sh = jnp.maximum(sh + bfc_ref[...], 0.0)
    z = jnp.dot(sh, wh_ref[...], preferred_element_type=_F32) + bh_ref[...]
    for s0, c in segments:
        zt = z[:, s0:s0 + c]
        m = jnp.max(zt, axis=1, keepdims=True)
        lse = jnp.log(jnp.sum(jnp.exp(zt - m), axis=1, keepdims=True))
        out_ref[:, s0:s0 + c] = zt - m - lse


_SEGMENTS = ((0, 10), (10, 5))
_CTOT = 15


@jax.jit
def _forward(w1, b1, g1, be1, w2, b2, g2, be2, wfc, bfc, wh, bh, x):
    n = x.shape[0]
    g2s, be2s = g2[:, :32], be2[:, :32]

    eye32, eye8 = jnp.eye(32, dtype=_F32), jnp.eye(8, dtype=_F32)
    w1bd = jnp.concatenate([
        jnp.concatenate([jnp.kron(eye32, w1[3 * t:3 * t + 3, :]),
                         jnp.zeros((32, 512), _F32)], axis=0)
        for t in range(9)], axis=0).astype(jnp.bfloat16)
    w2bd = jnp.concatenate([
        jnp.kron(eye8, w2[16 * t:16 * t + 16, :]) for t in range(9)],
        axis=0).astype(jnp.bfloat16)
    b1l = jnp.tile(b1, (1, 32))
    b2l = jnp.tile(b2, (1, 8))

    gr0 = 4
    x_l = pl.pallas_call(
        functools.partial(_transpose_in, groups=gr0),
        grid=(n // (32 * gr0),),
        in_specs=[pl.BlockSpec((96 * gr0, 1024), lambda i: (i, 0))],
        out_specs=pl.BlockSpec((1024, 128 * gr0), lambda i: (0, i)),
        out_shape=jax.ShapeDtypeStruct((1024, n * 4), jnp.bfloat16),
        compiler_params=pltpu.CompilerParams(
            dimension_semantics=("parallel",),
            vmem_limit_bytes=_VMEM_LIMIT),
    )(x.reshape(n * 3, 1024))

    gr1 = 4
    nb1 = n // (32 * gr1)
    pooled1, st1 = pl.pallas_call(
        functools.partial(_stage1, groups=gr1),
        grid=(nb1,),
        in_specs=[
            pl.BlockSpec((1024, 128 * gr1), lambda i: (0, i)),
            pl.BlockSpec((1152, 512), lambda i: (0, 0)),
            pl.BlockSpec((1, 512), lambda i: (0, 0)),
        ],
        out_specs=[
            pl.BlockSpec((256, 512 * gr1), lambda i: (0, i)),
            pl.BlockSpec((1, 2, 16), lambda i: (i, 0, 0)),
        ],
        out_shape=[
            jax.ShapeDtypeStruct((256, n * 16), jnp.bfloat16),
            jax.ShapeDtypeStruct((nb1, 2, 16), _F32),
        ],
        scratch_shapes=[pltpu.VMEM((1024, 128), _F32)] * 4,
        compiler_params=pltpu.CompilerParams(
            dimension_semantics=("parallel",),
            vmem_limit_bytes=_VMEM_LIMIT),
    )(x_l, w1bd, b1l)

    gr2 = 16
    nb2 = n // (8 * gr2)
    pooled2, st2 = pl.pallas_call(
        functools.partial(_stage2, groups=gr2, inv_m=1.0 / float(n * 225)),
        grid=(nb2,),
        in_specs=[
            pl.BlockSpec((256, 128 * gr2), lambda i: (0, i)),
            pl.BlockSpec((nb1, 2, 16), lambda i: (0, 0, 0)),
            pl.BlockSpec((1, 16), lambda i: (0, 0)),
            pl.BlockSpec((1, 16), lambda i: (0, 0)),
            pl.BlockSpec((1152, 256), lambda i: (0, 0)),
            pl.BlockSpec((1, 256), lambda i: (0, 0)),
        ],
        out_specs=[
            pl.BlockSpec((64, 256 * gr2), lambda i: (0, i)),
            pl.BlockSpec((1, 2, 32), lambda i: (i, 0, 0)),
        ],
        out_shape=[
            jax.ShapeDtypeStruct((64, n * 32), jnp.bfloat16),
            jax.ShapeDtypeStruct((nb2, 2, 32), _F32),
        ],
        scratch_shapes=[pltpu.VMEM((256, 128), _F32)] * 2,
        compiler_params=pltpu.CompilerParams(
            dimension_semantics=("parallel",),
            vmem_limit_bytes=_VMEM_LIMIT),
    )(pooled1, st1, g1, be1, w2bd, b2l)

    p4 = pooled2.reshape(8, 8, n, 32)[:6, :6]
    x3 = jnp.transpose(p4, (2, 0, 1, 3)).reshape(n, 1152)

    tb3 = min(512, n)
    logp = pl.pallas_call(
        functools.partial(_stage3, inv_m=1.0 / float(n * 36),
                          segments=_SEGMENTS),
        grid=(n // tb3,),
        in_specs=[
            pl.BlockSpec((tb3, 1152), lambda i: (i, 0)),
            pl.BlockSpec((nb2, 2, 32), lambda i: (0, 0, 0)),
            pl.BlockSpec((1, 32), lambda i: (0, 0)),
            pl.BlockSpec((1, 32), lambda i: (0, 0)),
            pl.BlockSpec((1152, 32), lambda i: (0, 0)),
            pl.BlockSpec((1, 32), lambda i: (0, 0)),
            pl.BlockSpec((32, _CTOT), lambda i: (0, 0)),
            pl.BlockSpec((1, _CTOT), lambda i: (0, 0)),
        ],
        out_specs=pl.BlockSpec((tb3, _CTOT), lambda i: (i, 0)),
        out_shape=jax.ShapeDtypeStruct((n, _CTOT), _F32),
        compiler_params=pltpu.CompilerParams(
            dimension_semantics=("parallel",),
            vmem_limit_bytes=_VMEM_LIMIT),
    )(x3, st2, g2s, be2s, wfc, bfc, wh, bh)

    return [logp[:, s:s + c] for s, c in _SEGMENTS]


def kernel(w1, b1, g1, be1, w2, b2, g2, be2, wfc, bfc, wh, bh, x):
    return _forward(w1, b1, g1, be1, w2, b2, g2, be2, wfc, bfc, wh, bh, x)


# speedup vs baseline: 8.8598x; 1.0979x over previous
import functools

import jax
import jax.numpy as jnp
from jax.experimental import pallas as pl
from jax.experimental.pallas import tpu as pltpu

_EPS = 1e-5
_VMEM_LIMIT = 64 * 1024 * 1024
_F32 = jnp.float32


def _iota_eq(rows, cols, mod):
    r = jax.lax.broadcasted_iota(jnp.int32, (rows, cols), 0)
    c = jax.lax.broadcasted_iota(jnp.int32, (rows, cols), 1)
    return (r % mod == c % mod).astype(_F32) if mod != cols else \
        (r % mod == c).astype(_F32)


def _transpose_in(x_ref, out_ref, *, groups):
    for g in range(groups):
        t = x_ref[96 * g:96 * (g + 1), :].astype(jnp.bfloat16).T
        out_ref[:, 128 * g:128 * g + 96] = t
        out_ref[:, 128 * g + 96:128 * (g + 1)] = jnp.zeros((1024, 32),
                                                           jnp.bfloat16)


def _stage1(x_ref, w_ref, b_ref, out_ref, st_ref, a0, a1, a2, a3, *,
            groups):
    accs = (a0, a1, a2, a3)
    ri = jax.lax.broadcasted_iota(jnp.int32, (240, 1), 0)
    vmask = (ri % 16 < 15).astype(_F32)
    fold = _iota_eq(512, 16, 16)
    st0 = None
    st1 = None
    for g in range(groups):
        for ck in range(8):
            r0 = 128 * ck
            nr = 128 if ck < 7 else 62
            acc = None
            for t in range(9):
                i, j = divmod(t, 3)
                off = i * 32 + j
                m = jnp.dot(x_ref[r0 + off:r0 + off + nr,
                                  128 * g:128 * (g + 1)],
                            w_ref[t * 128:(t + 1) * 128, :],
                            preferred_element_type=_F32)
                acc = m if acc is None else acc + m
            for k in range(4):
                accs[k][r0:r0 + nr, :] = acc[:, 128 * k:128 * (k + 1)]
        for k in range(4):
            accs[k][958:1024, :] = jnp.zeros((66, 128), _F32)

        for hc in range(2):
            for k in range(4):
                e = jnp.maximum(accs[k][pl.ds(512 * hc, 256, 2), :],
                                accs[k][pl.ds(512 * hc + 1, 256, 2), :])
                w4 = e.reshape(8, 2, 16, 128)
                hm = jnp.maximum(w4[:, 0], w4[:, 1]).reshape(128, 128)
                hm = hm + b_ref[:, 128 * k:128 * (k + 1)]
                out_ref[128 * hc:128 * hc + 128,
                        512 * g + 128 * k:512 * g + 128 * (k + 1)] = (
                    hm.astype(jnp.bfloat16))

        pv = out_ref[0:240, 512 * g:512 * (g + 1)].astype(_F32)
        pm = pv * vmask
        s0 = jnp.dot(jnp.sum(pm, axis=0, keepdims=True), fold,
                     preferred_element_type=_F32)
        s1 = jnp.dot(jnp.sum(pm * pm, axis=0, keepdims=True), fold,
                     preferred_element_type=_F32)
        st0 = s0 if st0 is None else st0 + s0
        st1 = s1 if st1 is None else st1 + s1
    st_ref[0, 0:1, :] = st0
    st_ref[0, 1:2, :] = st1


def _stage2(x_ref, sin_ref, g_ref, be_ref, w_ref, b_ref, out_ref, st_ref,
            a0, a1, *, groups, inv_m):
    s = jnp.sum(sin_ref[...], axis=0)
    mean = s[0:1, :] * inv_m
    var = jnp.maximum(s[1:2, :] * inv_m - mean * mean, 0.0)
    scale_c = jax.lax.rsqrt(var + _EPS) * g_ref[...]
    shift_c = be_ref[...] - mean * scale_c
    expand = _iota_eq(16, 128, 16)
    scale = jnp.dot(scale_c, expand, preferred_element_type=_F32)
    shift = jnp.dot(shift_c, expand, preferred_element_type=_F32)

    ri = jax.lax.broadcasted_iota(jnp.int32, (64, 1), 0)
    mask = ((ri % 8 < 6) & (ri // 8 < 6)).astype(_F32)
    fold = _iota_eq(128, 32, 32)
    accs = (a0, a1)
    st0 = None
    st1 = None
    for q in range(groups):
        a = jnp.maximum(x_ref[:, q * 128:(q + 1) * 128] * scale + shift, 0.0)
        ab = a.astype(jnp.bfloat16)
        acc = None
        for t in range(9):
            i, j = divmod(t, 3)
            off = i * 16 + j
            m = jnp.dot(ab[off:off + 222, :], w_ref[t * 128:(t + 1) * 128, :],
                        preferred_element_type=_F32)
            acc = m if acc is None else acc + m
        for k in range(2):
            accs[k][0:222, :] = acc[:, 128 * k:128 * (k + 1)]
            accs[k][222:256, :] = jnp.zeros((34, 128), _F32)
            e = jnp.maximum(accs[k][pl.ds(0, 128, 2), :],
                            accs[k][pl.ds(1, 128, 2), :])
            w4 = e.reshape(8, 2, 8, 128)
            hm = jnp.maximum(w4[:, 0], w4[:, 1]).reshape(64, 128)
            hm = hm + b_ref[:, 128 * k:128 * (k + 1)]
            hm = hm.astype(jnp.bfloat16)
            out_ref[:, q * 256 + 128 * k:q * 256 + 128 * (k + 1)] = hm
            pm = hm.astype(_F32) * mask
            s0 = jnp.dot(jnp.sum(pm, axis=0, keepdims=True), fold,
                         preferred_element_type=_F32)
            s1 = jnp.dot(jnp.sum(pm * pm, axis=0, keepdims=True), fold,
                         preferred_element_type=_F32)
            st0 = s0 if st0 is None else st0 + s0
            st1 = s1 if st1 is None else st1 + s1
    st_ref[0, 0:1, :] = st0
    st_ref[0, 1:2, :] = st1


def _stage3(x_ref, sin_ref, g_ref, be_ref, wfc_ref, bfc_ref, wh_ref, bh_ref,
            out_ref, *, inv_m, segments):
    s = jnp.sum(sin_ref[...], axis=0)
    mean = s[0:1, :] * inv_m
    var = jnp.maximum(s[1:2, :] * inv_m - mean * mean, 0.0)
    scale_c = jax.lax.rsqrt(var + _EPS) * g_ref[...]
    shift_c = be_ref[...] - mean * scale_c
    expand = _iota_eq(32, 2048, 32)
    scale = jnp.dot(scale_c, expand, preferred_element_type=_F32)
    shift = jnp.dot(shift_c, expand, preferred_element_type=_F32)
    act = jnp.maximum(x_ref[...] * scale + shift, 0.0)
    sh = jnp.dot(act.astype(jnp.bfloat16), wfc_ref[...],
                 preferred_element_type=_F32)
    sh = jnp.maximum(sh + bfc_ref[...], 0.0)
    z = jnp.dot(sh, wh_ref[...], preferred_element_type=_F32) + bh_ref[...]
    for s0, c in segments:
        zt = z[:, s0:s0 + c]
        m = jnp.max(zt, axis=1, keepdims=True)
        lse = jnp.log(jnp.sum(jnp.exp(zt - m), axis=1, keepdims=True))
        out_ref[:, s0:s0 + c] = zt - m - lse


_SEGMENTS = ((0, 10), (10, 5))
_CTOT = 15


@jax.jit
def _forward(w1, b1, g1, be1, w2, b2, g2, be2, wfc, bfc, wh, bh, x):
    n = x.shape[0]
    g2s, be2s = g2[:, :32], be2[:, :32]

    eye32, eye8 = jnp.eye(32, dtype=_F32), jnp.eye(8, dtype=_F32)
    w1bd = jnp.concatenate([
        jnp.concatenate([jnp.kron(eye32, w1[3 * t:3 * t + 3, :]),
                         jnp.zeros((32, 512), _F32)], axis=0)
        for t in range(9)], axis=0).astype(jnp.bfloat16)
    w2bd = jnp.concatenate([
        jnp.kron(eye8, w2[16 * t:16 * t + 16, :]) for t in range(9)],
        axis=0).astype(jnp.bfloat16)
    b1l = jnp.tile(b1, (1, 32))
    b2l = jnp.tile(b2, (1, 8))

    gr0 = 4
    x_l = pl.pallas_call(
        functools.partial(_transpose_in, groups=gr0),
        grid=(n // (32 * gr0),),
        in_specs=[pl.BlockSpec((96 * gr0, 1024), lambda i: (i, 0))],
        out_specs=pl.BlockSpec((1024, 128 * gr0), lambda i: (0, i)),
        out_shape=jax.ShapeDtypeStruct((1024, n * 4), jnp.bfloat16),
        compiler_params=pltpu.CompilerParams(
            dimension_semantics=("parallel",),
            vmem_limit_bytes=_VMEM_LIMIT),
    )(x.reshape(n * 3, 1024))

    gr1 = 4
    nb1 = n // (32 * gr1)
    pooled1, st1 = pl.pallas_call(
        functools.partial(_stage1, groups=gr1),
        grid=(nb1,),
        in_specs=[
            pl.BlockSpec((1024, 128 * gr1), lambda i: (0, i)),
            pl.BlockSpec((1152, 512), lambda i: (0, 0)),
            pl.BlockSpec((1, 512), lambda i: (0, 0)),
        ],
        out_specs=[
            pl.BlockSpec((256, 512 * gr1), lambda i: (0, i)),
            pl.BlockSpec((1, 2, 16), lambda i: (i, 0, 0)),
        ],
        out_shape=[
            jax.ShapeDtypeStruct((256, n * 16), jnp.bfloat16),
            jax.ShapeDtypeStruct((nb1, 2, 16), _F32),
        ],
        scratch_shapes=[pltpu.VMEM((1024, 128), _F32)] * 4,
        compiler_params=pltpu.CompilerParams(
            dimension_semantics=("parallel",),
            vmem_limit_bytes=_VMEM_LIMIT),
    )(x_l, w1bd, b1l)

    gr2 = 16
    nb2 = n // (8 * gr2)
    pooled2, st2 = pl.pallas_call(
        functools.partial(_stage2, groups=gr2, inv_m=1.0 / float(n * 225)),
        grid=(nb2,),
        in_specs=[
            pl.BlockSpec((256, 128 * gr2), lambda i: (0, i)),
            pl.BlockSpec((nb1, 2, 16), lambda i: (0, 0, 0)),
            pl.BlockSpec((1, 16), lambda i: (0, 0)),
            pl.BlockSpec((1, 16), lambda i: (0, 0)),
            pl.BlockSpec((1152, 256), lambda i: (0, 0)),
            pl.BlockSpec((1, 256), lambda i: (0, 0)),
        ],
        out_specs=[
            pl.BlockSpec((64, 256 * gr2), lambda i: (0, i)),
            pl.BlockSpec((1, 2, 32), lambda i: (i, 0, 0)),
        ],
        out_shape=[
            jax.ShapeDtypeStruct((64, n * 32), jnp.bfloat16),
            jax.ShapeDtypeStruct((nb2, 2, 32), _F32),
        ],
        scratch_shapes=[pltpu.VMEM((256, 128), _F32)] * 2,
        compiler_params=pltpu.CompilerParams(
            dimension_semantics=("parallel",),
            vmem_limit_bytes=_VMEM_LIMIT),
    )(pooled1, st1, g1, be1, w2bd, b2l)

    x3 = jnp.transpose(pooled2.reshape(64, n, 32), (1, 0, 2)).reshape(n, 2048)
    wfc_p = jnp.pad(wfc.reshape(6, 6, 32, 32),
                    ((0, 2), (0, 2), (0, 0), (0, 0))).reshape(2048, 32)
    wfc_p = wfc_p.astype(jnp.bfloat16)

    tb3 = min(512, n)
    logp = pl.pallas_call(
        functools.partial(_stage3, inv_m=1.0 / float(n * 36),
                          segments=_SEGMENTS),
        grid=(n // tb3,),
        in_specs=[
            pl.BlockSpec((tb3, 2048), lambda i: (i, 0)),
            pl.BlockSpec((nb2, 2, 32), lambda i: (0, 0, 0)),
            pl.BlockSpec((1, 32), lambda i: (0, 0)),
            pl.BlockSpec((1, 32), lambda i: (0, 0)),
            pl.BlockSpec((2048, 32), lambda i: (0, 0)),
            pl.BlockSpec((1, 32), lambda i: (0, 0)),
            pl.BlockSpec((32, _CTOT), lambda i: (0, 0)),
            pl.BlockSpec((1, _CTOT), lambda i: (0, 0)),
        ],
        out_specs=pl.BlockSpec((tb3, _CTOT), lambda i: (i, 0)),
        out_shape=jax.ShapeDtypeStruct((n, _CTOT), _F32),
        compiler_params=pltpu.CompilerParams(
            dimension_semantics=("parallel",),
            vmem_limit_bytes=_VMEM_LIMIT),
    )(x3, st2, g2s, be2s, wfc_p, bfc, wh, bh)

    return [logp[:, s:s + c] for s, c in _SEGMENTS]


def kernel(w1, b1, g1, be1, w2, b2, g2, be2, wfc, bfc, wh, bh, x):
    return _forward(w1, b1, g1, be1, w2, b2, g2, be2, wfc, bfc, wh, bh, x)
